# Initial kernel scaffold; baseline (speedup 1.0000x reference)
import jax
import jax.numpy as jnp
from jax.experimental import pallas as pl


def kernel(ids, weight):
    raise NotImplementedError("write your pallas kernel here")


# speedup vs baseline: 1.0944x; 1.0944x over previous
import functools

import jax
import jax.numpy as jnp
from jax import lax
from jax.experimental import pallas as pl
from jax.experimental.pallas import tpu as pltpu
from jax.experimental.pallas import tpu_sc as plsc


def kernel(ids, weight):
    B, S = ids.shape
    V, D = weight.shape
    N = B * S
    NC, NS = 2, 16
    NW = NC * NS
    n_per_w = N // NW
    C = 1024
    n_chunks = n_per_w // C

    flat_ids = ids.reshape(N)
    mesh = plsc.VectorSubcoreMesh(core_axis_name="c", subcore_axis_name="s")

    @functools.partial(
        pl.kernel,
        out_type=jax.ShapeDtypeStruct((N, D), jnp.float32),
        mesh=mesh,
        scratch_types=[
            pltpu.VMEM((C,), jnp.int32),
            pltpu.VMEM((C, D), jnp.float32),
            pltpu.SemaphoreType.DMA,
        ],
        compiler_params=pltpu.CompilerParams(use_tc_tiling_on_sc=False),
    )
    def emb(ids_hbm, w_hbm, out_hbm, idx_v, rows_v, sem):
        wid = lax.axis_index("s") * NC + lax.axis_index("c")
        base = wid * n_per_w

        def body(i, carry):
            off = base + i * C
            pltpu.sync_copy(ids_hbm.at[pl.ds(off, C)], idx_v)
            pltpu.async_copy(w_hbm.at[idx_v], rows_v, sem).wait()
            pltpu.sync_copy(rows_v, out_hbm.at[pl.ds(off, C)])
            return carry

        lax.fori_loop(0, n_chunks, body, 0)

    out = emb(flat_ids, weight)
    return out.reshape(B, S, D)


# speedup vs baseline: 1.1139x; 1.0178x over previous
import functools

import jax
import jax.numpy as jnp
from jax import lax
from jax.experimental import pallas as pl
from jax.experimental.pallas import tpu as pltpu
from jax.experimental.pallas import tpu_sc as plsc


def kernel(ids, weight):
    B, S = ids.shape
    V, D = weight.shape
    N = B * S
    NC, NS = 2, 16
    NW = NC * NS
    n_per_w = N // NW
    C = 1280
    n_chunks = n_per_w // C
    n_pairs = n_chunks // 2

    flat_ids = ids.reshape(N)
    mesh = plsc.VectorSubcoreMesh(core_axis_name="c", subcore_axis_name="s")

    @functools.partial(
        pl.kernel,
        out_type=jax.ShapeDtypeStruct((N, D), jnp.float32),
        mesh=mesh,
        scratch_types=[
            pltpu.VMEM((n_per_w,), jnp.int32),
            pltpu.VMEM((C, D), jnp.float32),
            pltpu.VMEM((C, D), jnp.float32),
            pltpu.SemaphoreType.DMA,
            pltpu.SemaphoreType.DMA,
        ],
        compiler_params=pltpu.CompilerParams(use_tc_tiling_on_sc=False),
    )
    def emb(ids_hbm, w_hbm, out_hbm, idx_v, rows0, rows1, sem0, sem1):
        wid = lax.axis_index("s") * NC + lax.axis_index("c")
        base = wid * n_per_w
        pltpu.sync_copy(ids_hbm.at[pl.ds(base, n_per_w)], idx_v)

        def gather(c, rows, sem):
            return pltpu.async_copy(w_hbm.at[idx_v.at[pl.ds(c * C, C)]], rows, sem)

        gather(0, rows0, sem0)

        def body(j, carry):
            c0 = 2 * j
            gather(c0 + 1, rows1, sem1)
            pltpu.make_async_copy(
                w_hbm.at[idx_v.at[pl.ds(c0 * C, C)]], rows0, sem0
            ).wait()
            pltpu.sync_copy(rows0, out_hbm.at[pl.ds(base + c0 * C, C)])

            @pl.when(j < n_pairs - 1)
            def _():
                gather(c0 + 2, rows0, sem0)

            pltpu.make_async_copy(
                w_hbm.at[idx_v.at[pl.ds((c0 + 1) * C, C)]], rows1, sem1
            ).wait()
            pltpu.sync_copy(rows1, out_hbm.at[pl.ds(base + (c0 + 1) * C, C)])
            return carry

        lax.fori_loop(0, n_pairs, body, 0)

    out = emb(flat_ids, weight)
    return out.reshape(B, S, D)


# speedup vs baseline: 1.6439x; 1.4758x over previous
import functools

import jax
import jax.numpy as jnp
from jax import lax
from jax.experimental import pallas as pl
from jax.experimental.pallas import tpu as pltpu
from jax.experimental.pallas import tpu_sc as plsc


def kernel(ids, weight):
    B, S = ids.shape
    V, D = weight.shape
    NC, NS = 2, 16
    NW = NC * NS
    BW = B // NW
    NBT = BW // 128
    n_pairs = S // 2

    ids_t = ids.T
    mesh = plsc.VectorSubcoreMesh(core_axis_name="c", subcore_axis_name="s")

    @functools.partial(
        pl.kernel,
        out_type=jax.ShapeDtypeStruct((S, D // 8, B // 128, 8, 128), jnp.float32),
        mesh=mesh,
        scratch_types=[
            pltpu.VMEM((S, BW), jnp.int32),
            pltpu.VMEM((BW, D), jnp.float32),
            pltpu.VMEM((BW, D), jnp.float32),
            pltpu.VMEM((D // 8, NBT, 8, 128), jnp.float32),
            pltpu.VMEM((D // 8, NBT, 8, 128), jnp.float32),
            pltpu.SemaphoreType.DMA,
            pltpu.SemaphoreType.DMA,
            pltpu.SemaphoreType.DMA,
            pltpu.SemaphoreType.DMA,
        ],
        compiler_params=pltpu.CompilerParams(
            use_tc_tiling_on_sc=False, needs_layout_passes=False),
    )
    def emb(ids_hbm, w_hbm, out_hbm, idx_v, rows0, rows1, slab0, slab1,
            sg0, sg1, ss0, ss1):
        wid = lax.axis_index("s") * NC + lax.axis_index("c")
        b0 = wid * BW
        bt0 = wid * NBT
        iota16 = lax.iota(jnp.int32, 16)
        cols = [jnp.full((16,), c, jnp.int32) for c in range(D)]
        pltpu.sync_copy(ids_hbm.at[:, pl.ds(b0, BW)], idx_v)

        def gather(s, rows, sem):
            return pltpu.async_copy(w_hbm.at[idx_v.at[s]], rows, sem)

        def transpose(rows, slab):
            def btl_body(btl, carry):
                for bg in range(8):
                    idx_b = btl * 128 + bg * 16 + iota16
                    for dt in range(D // 8):
                        for di in range(8):
                            v = plsc.load_gather(rows, [idx_b, cols[dt * 8 + di]])
                            slab[dt, btl, di, pl.ds(bg * 16, 16)] = v
                return carry

            lax.fori_loop(0, NBT, btl_body, 0)

        def slab_out(s, slab, sem):
            return pltpu.async_copy(slab, out_hbm.at[s, :, pl.ds(bt0, NBT)], sem)

        gather(0, rows0, sg0)

        def pair(j, carry):
            s0 = 2 * j
            gather(s0 + 1, rows1, sg1)
            pltpu.make_async_copy(w_hbm.at[idx_v.at[s0]], rows0, sg0).wait()

            @pl.when(j > 0)
            def _():
                pltpu.make_async_copy(slab0, out_hbm.at[s0 - 2, :, pl.ds(bt0, NBT)], ss0).wait()

            transpose(rows0, slab0)
            slab_out(s0, slab0, ss0)

            @pl.when(j < n_pairs - 1)
            def _():
                gather(s0 + 2, rows0, sg0)

            pltpu.make_async_copy(w_hbm.at[idx_v.at[s0 + 1]], rows1, sg1).wait()

            @pl.when(j > 0)
            def _():
                pltpu.make_async_copy(slab1, out_hbm.at[s0 - 1, :, pl.ds(bt0, NBT)], ss1).wait()

            transpose(rows1, slab1)
            slab_out(s0 + 1, slab1, ss1)
            return carry

        lax.fori_loop(0, n_pairs, pair, 0)
        pltpu.make_async_copy(slab0, out_hbm.at[S - 2, :, pl.ds(bt0, NBT)], ss0).wait()
        pltpu.make_async_copy(slab1, out_hbm.at[S - 1, :, pl.ds(bt0, NBT)], ss1).wait()

    out5 = emb(ids_t, weight)
    return out5.transpose(2, 4, 0, 1, 3).reshape(B, S, D)
